# baseline (speedup 1.0000x reference)
import functools

import jax
import jax.numpy as jnp
from jax import lax
from jax.experimental import pallas as pl
from jax.experimental.pallas import tpu as pltpu
from jax.experimental.pallas import tpu_sc as plsc

N_CORES = 2
N_SUBCORES = 16
N_TILES = N_CORES * N_SUBCORES
EDGE_CHUNK = 128
NBUF = 3
NIDX = 4
UNROLL = NBUF * NIDX
NBUF_DEC = 6
HPAD = 128

_MESH = plsc.VectorSubcoreMesh(
    core_axis_name="c", subcore_axis_name="s",
    num_cores=N_CORES, num_subcores=N_SUBCORES)


@functools.partial(jax.jit, static_argnames=("n_nodes", "d", "cpt"))
def _sc_aggregate(h, ei2, init_rows, *, n_nodes, d, cpt):
    rpt = (n_nodes // N_SUBCORES) // 8 * 8
    rem = n_nodes - rpt * N_SUBCORES

    @functools.partial(
        pl.kernel,
        out_type=jax.ShapeDtypeStruct((N_CORES, n_nodes, d), jnp.float32),
        mesh=_MESH,
        scratch_types=[
            pltpu.VMEM((NIDX, 2, EDGE_CHUNK), jnp.int32),
            pltpu.VMEM((NBUF, EDGE_CHUNK, d), jnp.float32),
            pltpu.VMEM_SHARED((n_nodes, d), jnp.float32),
            pltpu.SemaphoreType.DMA,
            pltpu.SemaphoreType.DMA,
            pltpu.SemaphoreType.DMA,
            pltpu.SemaphoreType.DMA,
            pltpu.SemaphoreType.DMA,
            pltpu.SemaphoreType.DMA,
            pltpu.SemaphoreType.DMA,
        ],
    )
    def agg_kernel(h_hbm, ei_hbm, init_hbm, out_hbm,
                   eib_v, rows_v, acc_sh,
                   i0, i1, i2, i3, g0, g1, g2):
        isems = (i0, i1, i2, i3)
        gsems = (g0, g1, g2)
        cid = lax.axis_index("c")
        sid = lax.axis_index("s")
        t0 = (cid * N_SUBCORES + sid) * cpt

        def idx_issue(c, s):
            pltpu.async_copy(ei_hbm.at[t0 + c], eib_v.at[s], isems[s])

        def idx_wait(c, s):
            pltpu.make_async_copy(ei_hbm.at[t0 + c], eib_v.at[s],
                                  isems[s]).wait()

        def gather(s, b):
            return pltpu.async_copy(h_hbm.at[eib_v.at[s, 0]], rows_v.at[b],
                                    gsems[b])

        def gather_wait(s, b):
            pltpu.make_async_copy(h_hbm.at[eib_v.at[s, 0]], rows_v.at[b],
                                  gsems[b]).wait()

        for s in range(NIDX):
            idx_issue(s, s)

        my_rows = pl.ds(sid * rpt, rpt)
        pltpu.sync_copy(init_hbm.at[cid, pl.ds(0, rpt)], acc_sh.at[my_rows])

        @pl.when(sid == N_SUBCORES - 1)
        def _init_tail():
            pltpu.sync_copy(
                init_hbm.at[cid, pl.ds(rpt, rem)],
                acc_sh.at[pl.ds(rpt * N_SUBCORES, rem)])

        plsc.subcore_barrier()

        for b in range(NBUF):
            idx_wait(b, b)
            gather(b, b)

        @pl.loop(0, cpt // UNROLL)
        def _ring(j):
            for k in range(UNROLL):
                c = j * UNROLL + k
                b = k % NBUF
                si = k % NIDX
                gather_wait(si, b)
                pltpu.sync_copy(rows_v.at[b], acc_sh.at[eib_v.at[si, 1]],
                                add=True)

                @pl.when(c + NBUF < cpt)
                def _rearm_gather():
                    idx_wait(c + NBUF, (k + NBUF) % NIDX)
                    gather((k + NBUF) % NIDX, b)

                @pl.when(c + NIDX < cpt)
                def _rearm_idx():
                    idx_issue(c + NIDX, si)

        plsc.subcore_barrier()
        pltpu.sync_copy(acc_sh.at[my_rows], out_hbm.at[cid, my_rows])

        @pl.when(sid == N_SUBCORES - 1)
        def _out_tail():
            tail = pl.ds(rpt * N_SUBCORES, rem)
            pltpu.sync_copy(acc_sh.at[tail], out_hbm.at[cid, tail])

    return agg_kernel(h, ei2, init_rows)


@functools.partial(jax.jit, static_argnames=("d", "cpt"))
def _sc_decode_gather(z, idx_l, *, d, cpt):
    n_out = cpt * N_TILES * EDGE_CHUNK

    @functools.partial(
        pl.kernel,
        out_type=jax.ShapeDtypeStruct((n_out, d), jnp.float32),
        mesh=_MESH,
        scratch_types=[
            pltpu.VMEM((cpt * EDGE_CHUNK,), jnp.int32),
            pltpu.VMEM((NBUF_DEC, EDGE_CHUNK, d), jnp.float32),
            pltpu.SemaphoreType.DMA,
            pltpu.SemaphoreType.DMA,
            pltpu.SemaphoreType.DMA,
            pltpu.SemaphoreType.DMA,
            pltpu.SemaphoreType.DMA,
            pltpu.SemaphoreType.DMA,
            pltpu.SemaphoreType.DMA,
        ],
    )
    def dec_kernel(z_hbm, idx_hbm, out_hbm,
                   idxb_v, rows_v,
                   isem, g0, g1, g2, g3, g4, g5):
        gsems = (g0, g1, g2, g3, g4, g5)
        cid = lax.axis_index("c")
        sid = lax.axis_index("s")
        t0 = (cid * N_SUBCORES + sid) * cpt * EDGE_CHUNK

        pltpu.async_copy(idx_hbm.at[pl.ds(t0, cpt * EDGE_CHUNK)],
                         idxb_v, isem).wait()

        def gather(c, b):
            return pltpu.async_copy(
                z_hbm.at[idxb_v.at[pl.ds(c * EDGE_CHUNK, EDGE_CHUNK)]],
                rows_v.at[b], gsems[b])

        for b in range(NBUF_DEC):
            gather(b, b)

        @pl.loop(0, cpt // NBUF_DEC)
        def _ring(j):
            base = j * NBUF_DEC
            for b in range(NBUF_DEC):
                c = base + b
                pltpu.make_async_copy(
                    z_hbm.at[idxb_v.at[pl.ds(c * EDGE_CHUNK, EDGE_CHUNK)]],
                    rows_v.at[b], gsems[b]).wait()
                sl = pl.ds(t0 + c * EDGE_CHUNK, EDGE_CHUNK)
                pltpu.sync_copy(rows_v.at[b], out_hbm.at[sl])

                @pl.when(c + NBUF_DEC < cpt)
                def _rearm():
                    gather(c + NBUF_DEC, b)

    return dec_kernel(z, idx_l)


def _mm_first(x, w):
    n, d = x.shape

    def body(x_ref, w_ref, o_ref):
        o_ref[pl.ds(0, n)] = jnp.dot(x_ref[...], w_ref[...],
                                     preferred_element_type=jnp.float32)
        o_ref[pl.ds(n, HPAD)] = jnp.zeros((HPAD, w_ref.shape[1]), jnp.float32)

    return pl.pallas_call(
        body,
        out_shape=jax.ShapeDtypeStruct((n + HPAD, w.shape[1]), jnp.float32),
    )(x, w)


def _mm_fused(parts, w):
    _, n, d = parts.shape

    def body(p_ref, w_ref, o_ref):
        z = jnp.maximum(p_ref[0] + p_ref[1], 0.0)
        o_ref[pl.ds(0, n)] = jnp.dot(z, w_ref[...],
                                     preferred_element_type=jnp.float32)
        o_ref[pl.ds(n, HPAD)] = jnp.zeros((HPAD, w_ref.shape[1]), jnp.float32)

    return pl.pallas_call(
        body,
        out_shape=jax.ShapeDtypeStruct((n + HPAD, w.shape[1]), jnp.float32),
    )(parts, w)


def _add_parts(parts):
    _, n, d = parts.shape

    def body(p_ref, o_ref):
        o_ref[...] = p_ref[0] + p_ref[1]

    return pl.pallas_call(
        body,
        out_shape=jax.ShapeDtypeStruct((n, d), jnp.float32),
    )(parts)


def _dot_rows(gs, gd):
    n, d = gs.shape
    grid = 20 if n % 160 == 0 else 16
    blk = n // grid

    def body(s_ref, d_ref, o_ref):
        o_ref[...] = jnp.sum(s_ref[...] * d_ref[...], axis=1, keepdims=True)

    return pl.pallas_call(
        body,
        grid=(grid,),
        in_specs=[pl.BlockSpec((blk, d), lambda i: (i, 0)),
                  pl.BlockSpec((blk, d), lambda i: (i, 0))],
        out_specs=pl.BlockSpec((blk, 1), lambda i: (i, 0)),
        out_shape=jax.ShapeDtypeStruct((n, 1), jnp.float32),
    )(gs, gd)


def kernel(x, edge_index, edge_label_index, W1, b1, W2, b2, W3, b3):
    n_nodes, d = x.shape
    n_edges = edge_index.shape[1]
    n_lbl = edge_label_index.shape[1]
    rpt = (n_nodes // N_SUBCORES) // 8 * 8
    rem = n_nodes - rpt * N_SUBCORES
    init_len = rpt + rem

    chunks_e = -(-n_edges // EDGE_CHUNK)
    cpt_e = -(-chunks_e // (N_TILES * UNROLL)) * UNROLL
    e_pad = cpt_e * N_TILES * EDGE_CHUNK
    n_fill = e_pad - n_edges
    fill = jnp.arange(n_fill, dtype=jnp.int32)
    src2 = jnp.concatenate([edge_index[0], n_nodes + fill % HPAD]
                           ).reshape(-1, EDGE_CHUNK)
    dst2 = jnp.concatenate([edge_index[1], fill % n_nodes]
                           ).reshape(-1, EDGE_CHUNK)
    ei2 = jnp.stack([src2, dst2], axis=1)

    def init_rows(b):
        return jnp.stack([
            jnp.broadcast_to(b, (init_len, d)),
            jnp.zeros((init_len, d), jnp.float32),
        ])

    h1 = _mm_first(x, W1)
    p1 = _sc_aggregate(h1, ei2, init_rows(b1),
                       n_nodes=n_nodes, d=d, cpt=cpt_e)
    h2 = _mm_fused(p1, W2)
    p2 = _sc_aggregate(h2, ei2, init_rows(b2),
                       n_nodes=n_nodes, d=d, cpt=cpt_e)
    h3 = _mm_fused(p2, W3)
    p3 = _sc_aggregate(h3, ei2, init_rows(b3),
                       n_nodes=n_nodes, d=d, cpt=cpt_e)
    z = _add_parts(p3)

    chunks_l = -(-(2 * n_lbl) // EDGE_CHUNK)
    cpt_l = -(-chunks_l // (N_TILES * NBUF_DEC)) * NBUF_DEC
    l_pad = cpt_l * N_TILES * EDGE_CHUNK
    fill_l = jnp.arange(l_pad - 2 * n_lbl, dtype=jnp.int32) % n_nodes
    idx_l = jnp.concatenate([edge_label_index[0], edge_label_index[1], fill_l])
    g = _sc_decode_gather(z, idx_l, d=d, cpt=cpt_l)
    dots = _dot_rows(g[:n_lbl], g[n_lbl:2 * n_lbl])
    return dots[:, 0]


# speedup vs baseline: 1.1208x; 1.1208x over previous
import functools

import jax
import jax.numpy as jnp
from jax import lax
from jax.experimental import pallas as pl
from jax.experimental.pallas import tpu as pltpu
from jax.experimental.pallas import tpu_sc as plsc

N_CORES = 2
N_SUBCORES = 16
N_TILES = N_CORES * N_SUBCORES
EDGE_CHUNK = 128
NBUF = 3
NIDX = 4
UNROLL = NBUF * NIDX
NBUF_DEC = 6
HPAD = 128

_MESH = plsc.VectorSubcoreMesh(
    core_axis_name="c", subcore_axis_name="s",
    num_cores=N_CORES, num_subcores=N_SUBCORES)


@functools.partial(jax.jit, static_argnames=("n_nodes", "d", "cpt"))
def _sc_aggregate(h, ei2, init_rows, *, n_nodes, d, cpt):
    rpt = (n_nodes // N_SUBCORES) // 8 * 8
    rem = n_nodes - rpt * N_SUBCORES

    @functools.partial(
        pl.kernel,
        out_type=jax.ShapeDtypeStruct((N_CORES, n_nodes, d), jnp.float32),
        mesh=_MESH,
        scratch_types=[
            pltpu.VMEM((NIDX, 2, EDGE_CHUNK), jnp.int32),
            pltpu.VMEM((NBUF, EDGE_CHUNK, d), jnp.float32),
            pltpu.VMEM_SHARED((n_nodes, d), jnp.float32),
            pltpu.SemaphoreType.DMA,
            pltpu.SemaphoreType.DMA,
            pltpu.SemaphoreType.DMA,
            pltpu.SemaphoreType.DMA,
            pltpu.SemaphoreType.DMA,
            pltpu.SemaphoreType.DMA,
            pltpu.SemaphoreType.DMA,
        ],
    )
    def agg_kernel(h_hbm, ei_hbm, init_hbm, out_hbm,
                   eib_v, rows_v, acc_sh,
                   i0, i1, i2, i3, g0, g1, g2):
        isems = (i0, i1, i2, i3)
        gsems = (g0, g1, g2)
        cid = lax.axis_index("c")
        sid = lax.axis_index("s")
        t0 = (cid * N_SUBCORES + sid) * cpt

        def idx_issue(c, s):
            pltpu.async_copy(ei_hbm.at[t0 + c], eib_v.at[s], isems[s])

        def idx_wait(c, s):
            pltpu.make_async_copy(ei_hbm.at[t0 + c], eib_v.at[s],
                                  isems[s]).wait()

        def gather(s, b):
            return pltpu.async_copy(h_hbm.at[eib_v.at[s, 0]], rows_v.at[b],
                                    gsems[b])

        def gather_wait(s, b):
            pltpu.make_async_copy(h_hbm.at[eib_v.at[s, 0]], rows_v.at[b],
                                  gsems[b]).wait()

        for s in range(NIDX):
            idx_issue(s, s)

        my_rows = pl.ds(sid * rpt, rpt)
        pltpu.sync_copy(init_hbm.at[cid, pl.ds(0, rpt)], acc_sh.at[my_rows])

        @pl.when(sid == N_SUBCORES - 1)
        def _init_tail():
            pltpu.sync_copy(
                init_hbm.at[cid, pl.ds(rpt, rem)],
                acc_sh.at[pl.ds(rpt * N_SUBCORES, rem)])

        plsc.subcore_barrier()

        for b in range(NBUF):
            idx_wait(b, b)
            gather(b, b)

        @pl.loop(0, cpt // UNROLL)
        def _ring(j):
            for k in range(UNROLL):
                c = j * UNROLL + k
                b = k % NBUF
                si = k % NIDX
                gather_wait(si, b)
                pltpu.sync_copy(rows_v.at[b], acc_sh.at[eib_v.at[si, 1]],
                                add=True)

                @pl.when(c + NBUF < cpt)
                def _rearm_gather():
                    idx_wait(c + NBUF, (k + NBUF) % NIDX)
                    gather((k + NBUF) % NIDX, b)

                @pl.when(c + NIDX < cpt)
                def _rearm_idx():
                    idx_issue(c + NIDX, si)

        plsc.subcore_barrier()
        pltpu.sync_copy(acc_sh.at[my_rows], out_hbm.at[cid, my_rows])

        @pl.when(sid == N_SUBCORES - 1)
        def _out_tail():
            tail = pl.ds(rpt * N_SUBCORES, rem)
            pltpu.sync_copy(acc_sh.at[tail], out_hbm.at[cid, tail])

    return agg_kernel(h, ei2, init_rows)


@functools.partial(jax.jit, static_argnames=("d", "cpt"))
def _sc_decode_gather(z, idx_l, *, d, cpt):
    n_out = cpt * N_TILES * EDGE_CHUNK

    @functools.partial(
        pl.kernel,
        out_type=jax.ShapeDtypeStruct((n_out, d), jnp.float32),
        mesh=_MESH,
        scratch_types=[
            pltpu.VMEM((cpt * EDGE_CHUNK,), jnp.int32),
            pltpu.VMEM((NBUF_DEC, EDGE_CHUNK, d), jnp.float32),
            pltpu.SemaphoreType.DMA,
            pltpu.SemaphoreType.DMA,
            pltpu.SemaphoreType.DMA,
            pltpu.SemaphoreType.DMA,
            pltpu.SemaphoreType.DMA,
            pltpu.SemaphoreType.DMA,
            pltpu.SemaphoreType.DMA,
        ],
    )
    def dec_kernel(z_hbm, idx_hbm, out_hbm,
                   idxb_v, rows_v,
                   isem, g0, g1, g2, g3, g4, g5):
        gsems = (g0, g1, g2, g3, g4, g5)
        cid = lax.axis_index("c")
        sid = lax.axis_index("s")
        t0 = (cid * N_SUBCORES + sid) * cpt * EDGE_CHUNK

        pltpu.async_copy(idx_hbm.at[pl.ds(t0, cpt * EDGE_CHUNK)],
                         idxb_v, isem).wait()

        def gather(c, b):
            return pltpu.async_copy(
                z_hbm.at[idxb_v.at[pl.ds(c * EDGE_CHUNK, EDGE_CHUNK)]],
                rows_v.at[b], gsems[b])

        for b in range(NBUF_DEC):
            gather(b, b)

        @pl.loop(0, cpt // NBUF_DEC)
        def _ring(j):
            base = j * NBUF_DEC
            for b in range(NBUF_DEC):
                c = base + b
                pltpu.make_async_copy(
                    z_hbm.at[idxb_v.at[pl.ds(c * EDGE_CHUNK, EDGE_CHUNK)]],
                    rows_v.at[b], gsems[b]).wait()
                sl = pl.ds(t0 + c * EDGE_CHUNK, EDGE_CHUNK)
                pltpu.sync_copy(rows_v.at[b], out_hbm.at[sl])

                @pl.when(c + NBUF_DEC < cpt)
                def _rearm():
                    gather(c + NBUF_DEC, b)

    return dec_kernel(z, idx_l)


def _mm_first(x, w):
    n, d = x.shape

    def body(x_ref, w_ref, o_ref):
        o_ref[pl.ds(0, n)] = jnp.dot(x_ref[...], w_ref[...],
                                     preferred_element_type=jnp.float32)
        o_ref[pl.ds(n, HPAD)] = jnp.zeros((HPAD, w_ref.shape[1]), jnp.float32)

    return pl.pallas_call(
        body,
        out_shape=jax.ShapeDtypeStruct((n + HPAD, w.shape[1]), jnp.float32),
    )(x, w)


def _mm_fused(parts, w):
    _, n, d = parts.shape

    def body(p_ref, w_ref, o_ref):
        z = jnp.maximum(p_ref[0] + p_ref[1], 0.0)
        o_ref[pl.ds(0, n)] = jnp.dot(z, w_ref[...],
                                     preferred_element_type=jnp.float32)
        o_ref[pl.ds(n, HPAD)] = jnp.zeros((HPAD, w_ref.shape[1]), jnp.float32)

    return pl.pallas_call(
        body,
        out_shape=jax.ShapeDtypeStruct((n + HPAD, w.shape[1]), jnp.float32),
    )(parts, w)


def _add_parts(parts):
    _, n, d = parts.shape

    def body(p_ref, o_ref):
        o_ref[...] = p_ref[0] + p_ref[1]

    return pl.pallas_call(
        body,
        out_shape=jax.ShapeDtypeStruct((n, d), jnp.float32),
    )(parts)


def _dot_rows(g, n):
    d = g.shape[1]
    grid = 20 if n % 160 == 0 else 16
    blk = n // grid

    def body(s_ref, d_ref, o_ref):
        o_ref[...] = jnp.sum(s_ref[...] * d_ref[...], axis=1, keepdims=True)

    return pl.pallas_call(
        body,
        grid=(grid,),
        in_specs=[pl.BlockSpec((blk, d), lambda i: (i, 0)),
                  pl.BlockSpec((blk, d), lambda i: (i + grid, 0))],
        out_specs=pl.BlockSpec((blk, 1), lambda i: (i, 0)),
        out_shape=jax.ShapeDtypeStruct((n, 1), jnp.float32),
    )(g, g)


def kernel(x, edge_index, edge_label_index, W1, b1, W2, b2, W3, b3):
    n_nodes, d = x.shape
    n_edges = edge_index.shape[1]
    n_lbl = edge_label_index.shape[1]
    rpt = (n_nodes // N_SUBCORES) // 8 * 8
    rem = n_nodes - rpt * N_SUBCORES
    init_len = rpt + rem

    chunks_e = -(-n_edges // EDGE_CHUNK)
    cpt_e = -(-chunks_e // (N_TILES * UNROLL)) * UNROLL
    e_pad = cpt_e * N_TILES * EDGE_CHUNK
    n_fill = e_pad - n_edges
    fill = jnp.arange(n_fill, dtype=jnp.int32)
    src2 = jnp.concatenate([edge_index[0], n_nodes + fill % HPAD]
                           ).reshape(-1, EDGE_CHUNK)
    dst2 = jnp.concatenate([edge_index[1], fill % n_nodes]
                           ).reshape(-1, EDGE_CHUNK)
    ei2 = jnp.stack([src2, dst2], axis=1)

    def init_rows(b):
        return jnp.stack([
            jnp.broadcast_to(b, (init_len, d)),
            jnp.zeros((init_len, d), jnp.float32),
        ])

    h1 = _mm_first(x, W1)
    p1 = _sc_aggregate(h1, ei2, init_rows(b1),
                       n_nodes=n_nodes, d=d, cpt=cpt_e)
    h2 = _mm_fused(p1, W2)
    p2 = _sc_aggregate(h2, ei2, init_rows(b2),
                       n_nodes=n_nodes, d=d, cpt=cpt_e)
    h3 = _mm_fused(p2, W3)
    p3 = _sc_aggregate(h3, ei2, init_rows(b3),
                       n_nodes=n_nodes, d=d, cpt=cpt_e)
    z = _add_parts(p3)

    chunks_l = -(-(2 * n_lbl) // EDGE_CHUNK)
    cpt_l = -(-chunks_l // (N_TILES * NBUF_DEC)) * NBUF_DEC
    l_pad = cpt_l * N_TILES * EDGE_CHUNK
    fill_l = jnp.arange(l_pad - 2 * n_lbl, dtype=jnp.int32) % n_nodes
    idx_l = jnp.concatenate([edge_label_index[0], edge_label_index[1], fill_l])
    g = _sc_decode_gather(z, idx_l, d=d, cpt=cpt_l)
    dots = _dot_rows(g, n_lbl)
    return dots[:, 0]


# speedup vs baseline: 1.1603x; 1.0352x over previous
import functools

import jax
import jax.numpy as jnp
from jax import lax
from jax.experimental import pallas as pl
from jax.experimental.pallas import tpu as pltpu
from jax.experimental.pallas import tpu_sc as plsc

N_CORES = 2
N_SUBCORES = 16
N_TILES = N_CORES * N_SUBCORES
EDGE_CHUNK = 128
AGG_CHUNK = 120
NBUF = 3
NIDX = 4
UNROLL = NBUF * NIDX
NBUF_DEC = 7
HPAD = 128

_MESH = plsc.VectorSubcoreMesh(
    core_axis_name="c", subcore_axis_name="s",
    num_cores=N_CORES, num_subcores=N_SUBCORES)


@functools.partial(jax.jit, static_argnames=("n_nodes", "d", "cpt"))
def _sc_aggregate(h, ei2, init_rows, *, n_nodes, d, cpt):
    rpt = (n_nodes // N_SUBCORES) // 8 * 8
    rem = n_nodes - rpt * N_SUBCORES

    @functools.partial(
        pl.kernel,
        out_type=jax.ShapeDtypeStruct((N_CORES, n_nodes, d), jnp.float32),
        mesh=_MESH,
        scratch_types=[
            pltpu.VMEM((NIDX, 2, AGG_CHUNK), jnp.int32),
            pltpu.VMEM((NBUF, AGG_CHUNK, d), jnp.float32),
            pltpu.VMEM_SHARED((n_nodes, d), jnp.float32),
            pltpu.SemaphoreType.DMA,
            pltpu.SemaphoreType.DMA,
            pltpu.SemaphoreType.DMA,
            pltpu.SemaphoreType.DMA,
            pltpu.SemaphoreType.DMA,
            pltpu.SemaphoreType.DMA,
            pltpu.SemaphoreType.DMA,
        ],
    )
    def agg_kernel(h_hbm, ei_hbm, init_hbm, out_hbm,
                   eib_v, rows_v, acc_sh,
                   i0, i1, i2, i3, g0, g1, g2):
        isems = (i0, i1, i2, i3)
        gsems = (g0, g1, g2)
        cid = lax.axis_index("c")
        sid = lax.axis_index("s")
        t0 = (cid * N_SUBCORES + sid) * cpt

        def idx_issue(c, s):
            pltpu.async_copy(ei_hbm.at[t0 + c], eib_v.at[s], isems[s])

        def idx_wait(c, s):
            pltpu.make_async_copy(ei_hbm.at[t0 + c], eib_v.at[s],
                                  isems[s]).wait()

        def gather(s, b):
            return pltpu.async_copy(h_hbm.at[eib_v.at[s, 0]], rows_v.at[b],
                                    gsems[b])

        def gather_wait(s, b):
            pltpu.make_async_copy(h_hbm.at[eib_v.at[s, 0]], rows_v.at[b],
                                  gsems[b]).wait()

        for s in range(NIDX):
            idx_issue(s, s)

        my_rows = pl.ds(sid * rpt, rpt)
        pltpu.sync_copy(init_hbm.at[cid, pl.ds(0, rpt)], acc_sh.at[my_rows])

        @pl.when(sid == N_SUBCORES - 1)
        def _init_tail():
            pltpu.sync_copy(
                init_hbm.at[cid, pl.ds(rpt, rem)],
                acc_sh.at[pl.ds(rpt * N_SUBCORES, rem)])

        plsc.subcore_barrier()

        for b in range(NBUF):
            idx_wait(b, b)
            gather(b, b)

        @pl.loop(0, cpt // UNROLL)
        def _ring(j):
            for k in range(UNROLL):
                c = j * UNROLL + k
                b = k % NBUF
                si = k % NIDX
                gather_wait(si, b)
                pltpu.sync_copy(rows_v.at[b], acc_sh.at[eib_v.at[si, 1]],
                                add=True)

                @pl.when(c + NBUF < cpt)
                def _rearm_gather():
                    idx_wait(c + NBUF, (k + NBUF) % NIDX)
                    gather((k + NBUF) % NIDX, b)

                @pl.when(c + NIDX < cpt)
                def _rearm_idx():
                    idx_issue(c + NIDX, si)

        plsc.subcore_barrier()
        pltpu.sync_copy(acc_sh.at[my_rows], out_hbm.at[cid, my_rows])

        @pl.when(sid == N_SUBCORES - 1)
        def _out_tail():
            tail = pl.ds(rpt * N_SUBCORES, rem)
            pltpu.sync_copy(acc_sh.at[tail], out_hbm.at[cid, tail])

    return agg_kernel(h, ei2, init_rows)


@functools.partial(jax.jit, static_argnames=("d", "cpt"))
def _sc_decode_gather(z, idx_l, *, d, cpt):
    n_out = cpt * N_TILES * EDGE_CHUNK

    @functools.partial(
        pl.kernel,
        out_type=jax.ShapeDtypeStruct((n_out, d), jnp.float32),
        mesh=_MESH,
        scratch_types=[
            pltpu.VMEM((cpt * EDGE_CHUNK,), jnp.int32),
            pltpu.VMEM((NBUF_DEC, EDGE_CHUNK, d), jnp.float32),
            pltpu.SemaphoreType.DMA,
            pltpu.SemaphoreType.DMA,
            pltpu.SemaphoreType.DMA,
            pltpu.SemaphoreType.DMA,
            pltpu.SemaphoreType.DMA,
            pltpu.SemaphoreType.DMA,
            pltpu.SemaphoreType.DMA,
            pltpu.SemaphoreType.DMA,
        ],
    )
    def dec_kernel(z_hbm, idx_hbm, out_hbm,
                   idxb_v, rows_v,
                   isem, g0, g1, g2, g3, g4, g5, g6):
        gsems = (g0, g1, g2, g3, g4, g5, g6)
        cid = lax.axis_index("c")
        sid = lax.axis_index("s")
        t0 = (cid * N_SUBCORES + sid) * cpt * EDGE_CHUNK

        pltpu.async_copy(idx_hbm.at[pl.ds(t0, cpt * EDGE_CHUNK)],
                         idxb_v, isem).wait()

        def gather(c, b):
            return pltpu.async_copy(
                z_hbm.at[idxb_v.at[pl.ds(c * EDGE_CHUNK, EDGE_CHUNK)]],
                rows_v.at[b], gsems[b])

        for b in range(NBUF_DEC):
            gather(b, b)

        @pl.loop(0, cpt // NBUF_DEC)
        def _ring(j):
            base = j * NBUF_DEC
            for b in range(NBUF_DEC):
                c = base + b
                pltpu.make_async_copy(
                    z_hbm.at[idxb_v.at[pl.ds(c * EDGE_CHUNK, EDGE_CHUNK)]],
                    rows_v.at[b], gsems[b]).wait()
                sl = pl.ds(t0 + c * EDGE_CHUNK, EDGE_CHUNK)
                pltpu.sync_copy(rows_v.at[b], out_hbm.at[sl])

                @pl.when(c + NBUF_DEC < cpt)
                def _rearm():
                    gather(c + NBUF_DEC, b)

    return dec_kernel(z, idx_l)


def _mm_first(x, w):
    n, d = x.shape

    def body(x_ref, w_ref, o_ref):
        o_ref[pl.ds(0, n)] = jnp.dot(x_ref[...], w_ref[...],
                                     preferred_element_type=jnp.float32)
        o_ref[pl.ds(n, HPAD)] = jnp.zeros((HPAD, w_ref.shape[1]), jnp.float32)

    return pl.pallas_call(
        body,
        out_shape=jax.ShapeDtypeStruct((n + HPAD, w.shape[1]), jnp.float32),
    )(x, w)


def _mm_fused(parts, w):
    _, n, d = parts.shape

    def body(p_ref, w_ref, o_ref):
        z = jnp.maximum(p_ref[0] + p_ref[1], 0.0)
        o_ref[pl.ds(0, n)] = jnp.dot(z, w_ref[...],
                                     preferred_element_type=jnp.float32)
        o_ref[pl.ds(n, HPAD)] = jnp.zeros((HPAD, w_ref.shape[1]), jnp.float32)

    return pl.pallas_call(
        body,
        out_shape=jax.ShapeDtypeStruct((n + HPAD, w.shape[1]), jnp.float32),
    )(parts, w)


def _add_parts(parts):
    _, n, d = parts.shape

    def body(p_ref, o_ref):
        o_ref[...] = p_ref[0] + p_ref[1]

    return pl.pallas_call(
        body,
        out_shape=jax.ShapeDtypeStruct((n, d), jnp.float32),
    )(parts)


def _dot_rows(g, n):
    d = g.shape[1]
    grid = 20 if n % 160 == 0 else 16
    blk = n // grid

    def body(s_ref, d_ref, o_ref):
        o_ref[...] = jnp.sum(s_ref[...] * d_ref[...], axis=1, keepdims=True)

    return pl.pallas_call(
        body,
        grid=(grid,),
        in_specs=[pl.BlockSpec((blk, d), lambda i: (i, 0)),
                  pl.BlockSpec((blk, d), lambda i: (i + grid, 0))],
        out_specs=pl.BlockSpec((blk, 1), lambda i: (i, 0)),
        out_shape=jax.ShapeDtypeStruct((n, 1), jnp.float32),
    )(g, g)


def kernel(x, edge_index, edge_label_index, W1, b1, W2, b2, W3, b3):
    n_nodes, d = x.shape
    n_edges = edge_index.shape[1]
    n_lbl = edge_label_index.shape[1]
    rpt = (n_nodes // N_SUBCORES) // 8 * 8
    rem = n_nodes - rpt * N_SUBCORES
    init_len = rpt + rem

    chunks_e = -(-n_edges // AGG_CHUNK)
    cpt_e = -(-chunks_e // (N_TILES * UNROLL)) * UNROLL
    e_pad = cpt_e * N_TILES * AGG_CHUNK
    n_fill = e_pad - n_edges
    fill = jnp.arange(n_fill, dtype=jnp.int32)
    src2 = jnp.concatenate([edge_index[0], n_nodes + fill % HPAD]
                           ).reshape(-1, AGG_CHUNK)
    dst2 = jnp.concatenate([edge_index[1], fill % n_nodes]
                           ).reshape(-1, AGG_CHUNK)
    ei2 = jnp.stack([src2, dst2], axis=1)

    def init_rows(b):
        return jnp.stack([
            jnp.broadcast_to(b, (init_len, d)),
            jnp.zeros((init_len, d), jnp.float32),
        ])

    h1 = _mm_first(x, W1)
    p1 = _sc_aggregate(h1, ei2, init_rows(b1),
                       n_nodes=n_nodes, d=d, cpt=cpt_e)
    h2 = _mm_fused(p1, W2)
    p2 = _sc_aggregate(h2, ei2, init_rows(b2),
                       n_nodes=n_nodes, d=d, cpt=cpt_e)
    h3 = _mm_fused(p2, W3)
    p3 = _sc_aggregate(h3, ei2, init_rows(b3),
                       n_nodes=n_nodes, d=d, cpt=cpt_e)
    z = _add_parts(p3)

    chunks_l = -(-(2 * n_lbl) // EDGE_CHUNK)
    cpt_l = -(-chunks_l // (N_TILES * NBUF_DEC)) * NBUF_DEC
    l_pad = cpt_l * N_TILES * EDGE_CHUNK
    fill_l = jnp.arange(l_pad - 2 * n_lbl, dtype=jnp.int32) % n_nodes
    idx_l = jnp.concatenate([edge_label_index[0], edge_label_index[1], fill_l])
    g = _sc_decode_gather(z, idx_l, d=d, cpt=cpt_l)
    dots = _dot_rows(g, n_lbl)
    return dots[:, 0]


# speedup vs baseline: 1.1634x; 1.0027x over previous
import functools

import jax
import jax.numpy as jnp
from jax import lax
from jax.experimental import pallas as pl
from jax.experimental.pallas import tpu as pltpu
from jax.experimental.pallas import tpu_sc as plsc

N_CORES = 2
N_SUBCORES = 16
N_TILES = N_CORES * N_SUBCORES
EDGE_CHUNK = 128
AGG_CHUNK = 120
NBUF = 3
NIDX = 6
UNROLL = 6
NBUF_DEC = 7
HPAD = 128

_MESH = plsc.VectorSubcoreMesh(
    core_axis_name="c", subcore_axis_name="s",
    num_cores=N_CORES, num_subcores=N_SUBCORES)


@functools.partial(jax.jit, static_argnames=("n_nodes", "d", "cpt"))
def _sc_aggregate(h, ei2, init_rows, *, n_nodes, d, cpt):
    rpt = (n_nodes // N_SUBCORES) // 8 * 8
    rem = n_nodes - rpt * N_SUBCORES

    @functools.partial(
        pl.kernel,
        out_type=jax.ShapeDtypeStruct((N_CORES, n_nodes, d), jnp.float32),
        mesh=_MESH,
        scratch_types=[
            pltpu.VMEM((NIDX, 2, AGG_CHUNK), jnp.int32),
            pltpu.VMEM((NBUF, AGG_CHUNK, d), jnp.float32),
            pltpu.VMEM_SHARED((n_nodes, d), jnp.float32),
            pltpu.SemaphoreType.DMA,
            pltpu.SemaphoreType.DMA,
            pltpu.SemaphoreType.DMA,
            pltpu.SemaphoreType.DMA,
            pltpu.SemaphoreType.DMA,
            pltpu.SemaphoreType.DMA,
            pltpu.SemaphoreType.DMA,
            pltpu.SemaphoreType.DMA,
            pltpu.SemaphoreType.DMA,
        ],
    )
    def agg_kernel(h_hbm, ei_hbm, init_hbm, out_hbm,
                   eib_v, rows_v, acc_sh,
                   i0, i1, i2, i3, i4, i5, g0, g1, g2):
        isems = (i0, i1, i2, i3, i4, i5)
        gsems = (g0, g1, g2)
        cid = lax.axis_index("c")
        sid = lax.axis_index("s")
        t0 = (cid * N_SUBCORES + sid) * cpt

        def idx_issue(c, s):
            pltpu.async_copy(ei_hbm.at[t0 + c], eib_v.at[s], isems[s])

        def idx_wait(c, s):
            pltpu.make_async_copy(ei_hbm.at[t0 + c], eib_v.at[s],
                                  isems[s]).wait()

        def gather(s, b):
            return pltpu.async_copy(h_hbm.at[eib_v.at[s, 0]], rows_v.at[b],
                                    gsems[b])

        def gather_wait(s, b):
            pltpu.make_async_copy(h_hbm.at[eib_v.at[s, 0]], rows_v.at[b],
                                  gsems[b]).wait()

        for s in range(NIDX):
            idx_issue(s, s)

        my_rows = pl.ds(sid * rpt, rpt)
        pltpu.sync_copy(init_hbm.at[cid, pl.ds(0, rpt)], acc_sh.at[my_rows])

        @pl.when(sid == N_SUBCORES - 1)
        def _init_tail():
            pltpu.sync_copy(
                init_hbm.at[cid, pl.ds(rpt, rem)],
                acc_sh.at[pl.ds(rpt * N_SUBCORES, rem)])

        plsc.subcore_barrier()

        for b in range(NBUF):
            idx_wait(b, b)
            gather(b, b)

        @pl.loop(0, cpt // UNROLL)
        def _ring(j):
            for k in range(UNROLL):
                c = j * UNROLL + k
                b = k % NBUF
                si = k % NIDX
                gather_wait(si, b)
                pltpu.sync_copy(rows_v.at[b], acc_sh.at[eib_v.at[si, 1]],
                                add=True)

                @pl.when(c + NBUF < cpt)
                def _rearm_gather():
                    idx_wait(c + NBUF, (k + NBUF) % NIDX)
                    gather((k + NBUF) % NIDX, b)

                @pl.when(c + NIDX < cpt)
                def _rearm_idx():
                    idx_issue(c + NIDX, si)

        plsc.subcore_barrier()
        pltpu.sync_copy(acc_sh.at[my_rows], out_hbm.at[cid, my_rows])

        @pl.when(sid == N_SUBCORES - 1)
        def _out_tail():
            tail = pl.ds(rpt * N_SUBCORES, rem)
            pltpu.sync_copy(acc_sh.at[tail], out_hbm.at[cid, tail])

    return agg_kernel(h, ei2, init_rows)


@functools.partial(jax.jit, static_argnames=("d", "cpt"))
def _sc_decode_gather(z, idx_l, *, d, cpt):
    n_out = cpt * N_TILES * EDGE_CHUNK

    @functools.partial(
        pl.kernel,
        out_type=jax.ShapeDtypeStruct((n_out, d), jnp.float32),
        mesh=_MESH,
        scratch_types=[
            pltpu.VMEM((cpt * EDGE_CHUNK,), jnp.int32),
            pltpu.VMEM((NBUF_DEC, EDGE_CHUNK, d), jnp.float32),
            pltpu.SemaphoreType.DMA,
            pltpu.SemaphoreType.DMA,
            pltpu.SemaphoreType.DMA,
            pltpu.SemaphoreType.DMA,
            pltpu.SemaphoreType.DMA,
            pltpu.SemaphoreType.DMA,
            pltpu.SemaphoreType.DMA,
            pltpu.SemaphoreType.DMA,
        ],
    )
    def dec_kernel(z_hbm, idx_hbm, out_hbm,
                   idxb_v, rows_v,
                   isem, g0, g1, g2, g3, g4, g5, g6):
        gsems = (g0, g1, g2, g3, g4, g5, g6)
        cid = lax.axis_index("c")
        sid = lax.axis_index("s")
        t0 = (cid * N_SUBCORES + sid) * cpt * EDGE_CHUNK

        pltpu.async_copy(idx_hbm.at[pl.ds(t0, cpt * EDGE_CHUNK)],
                         idxb_v, isem).wait()

        def gather(c, b):
            return pltpu.async_copy(
                z_hbm.at[idxb_v.at[pl.ds(c * EDGE_CHUNK, EDGE_CHUNK)]],
                rows_v.at[b], gsems[b])

        for b in range(NBUF_DEC):
            gather(b, b)

        @pl.loop(0, cpt // NBUF_DEC)
        def _ring(j):
            base = j * NBUF_DEC
            for b in range(NBUF_DEC):
                c = base + b
                pltpu.make_async_copy(
                    z_hbm.at[idxb_v.at[pl.ds(c * EDGE_CHUNK, EDGE_CHUNK)]],
                    rows_v.at[b], gsems[b]).wait()
                sl = pl.ds(t0 + c * EDGE_CHUNK, EDGE_CHUNK)
                pltpu.sync_copy(rows_v.at[b], out_hbm.at[sl])

                @pl.when(c + NBUF_DEC < cpt)
                def _rearm():
                    gather(c + NBUF_DEC, b)

    return dec_kernel(z, idx_l)


def _mm_first(x, w):
    n, d = x.shape

    def body(x_ref, w_ref, o_ref):
        o_ref[pl.ds(0, n)] = jnp.dot(x_ref[...], w_ref[...],
                                     preferred_element_type=jnp.float32)
        o_ref[pl.ds(n, HPAD)] = jnp.zeros((HPAD, w_ref.shape[1]), jnp.float32)

    return pl.pallas_call(
        body,
        out_shape=jax.ShapeDtypeStruct((n + HPAD, w.shape[1]), jnp.float32),
    )(x, w)


def _mm_fused(parts, w):
    _, n, d = parts.shape

    def body(p_ref, w_ref, o_ref):
        z = jnp.maximum(p_ref[0] + p_ref[1], 0.0)
        o_ref[pl.ds(0, n)] = jnp.dot(z, w_ref[...],
                                     preferred_element_type=jnp.float32)
        o_ref[pl.ds(n, HPAD)] = jnp.zeros((HPAD, w_ref.shape[1]), jnp.float32)

    return pl.pallas_call(
        body,
        out_shape=jax.ShapeDtypeStruct((n + HPAD, w.shape[1]), jnp.float32),
    )(parts, w)


def _add_parts(parts):
    _, n, d = parts.shape

    def body(p_ref, o_ref):
        o_ref[...] = p_ref[0] + p_ref[1]

    return pl.pallas_call(
        body,
        out_shape=jax.ShapeDtypeStruct((n, d), jnp.float32),
    )(parts)


def _dot_rows(g, n):
    d = g.shape[1]
    grid = 20 if n % 160 == 0 else 16
    blk = n // grid

    def body(s_ref, d_ref, o_ref):
        o_ref[...] = jnp.sum(s_ref[...] * d_ref[...], axis=1, keepdims=True)

    return pl.pallas_call(
        body,
        grid=(grid,),
        in_specs=[pl.BlockSpec((blk, d), lambda i: (i, 0)),
                  pl.BlockSpec((blk, d), lambda i: (i + grid, 0))],
        out_specs=pl.BlockSpec((blk, 1), lambda i: (i, 0)),
        out_shape=jax.ShapeDtypeStruct((n, 1), jnp.float32),
    )(g, g)


def kernel(x, edge_index, edge_label_index, W1, b1, W2, b2, W3, b3):
    n_nodes, d = x.shape
    n_edges = edge_index.shape[1]
    n_lbl = edge_label_index.shape[1]
    rpt = (n_nodes // N_SUBCORES) // 8 * 8
    rem = n_nodes - rpt * N_SUBCORES
    init_len = rpt + rem

    chunks_e = -(-n_edges // AGG_CHUNK)
    cpt_e = -(-chunks_e // (N_TILES * UNROLL)) * UNROLL
    e_pad = cpt_e * N_TILES * AGG_CHUNK
    n_fill = e_pad - n_edges
    fill = jnp.arange(n_fill, dtype=jnp.int32)
    src2 = jnp.concatenate([edge_index[0], n_nodes + fill % HPAD]
                           ).reshape(-1, AGG_CHUNK)
    dst2 = jnp.concatenate([edge_index[1], fill % n_nodes]
                           ).reshape(-1, AGG_CHUNK)
    ei2 = jnp.stack([src2, dst2], axis=1)

    def init_rows(b):
        return jnp.stack([
            jnp.broadcast_to(b, (init_len, d)),
            jnp.zeros((init_len, d), jnp.float32),
        ])

    h1 = _mm_first(x, W1)
    p1 = _sc_aggregate(h1, ei2, init_rows(b1),
                       n_nodes=n_nodes, d=d, cpt=cpt_e)
    h2 = _mm_fused(p1, W2)
    p2 = _sc_aggregate(h2, ei2, init_rows(b2),
                       n_nodes=n_nodes, d=d, cpt=cpt_e)
    h3 = _mm_fused(p2, W3)
    p3 = _sc_aggregate(h3, ei2, init_rows(b3),
                       n_nodes=n_nodes, d=d, cpt=cpt_e)
    z = _add_parts(p3)

    chunks_l = -(-(2 * n_lbl) // EDGE_CHUNK)
    cpt_l = -(-chunks_l // (N_TILES * NBUF_DEC)) * NBUF_DEC
    l_pad = cpt_l * N_TILES * EDGE_CHUNK
    fill_l = jnp.arange(l_pad - 2 * n_lbl, dtype=jnp.int32) % n_nodes
    idx_l = jnp.concatenate([edge_label_index[0], edge_label_index[1], fill_l])
    g = _sc_decode_gather(z, idx_l, d=d, cpt=cpt_l)
    dots = _dot_rows(g, n_lbl)
    return dots[:, 0]
